# baseline (speedup 1.0000x reference)
import jax
import jax.numpy as jnp
from jax import lax
from jax.experimental import pallas as pl
from jax.experimental.pallas import tpu as pltpu
from jax.experimental.pallas import tpu_sc as plsc

_B = 100000
_D = 256
_CH = 200
_NCHUNKS = _B // _CH
_ITERS = -(-_NCHUNKS // 32)
_NRB = 2
_NIB = 4

_INFO = plsc.get_sparse_core_info()
_NC = _INFO.num_cores
_NW = _INFO.num_cores * _INFO.num_subcores

_MESH = plsc.VectorSubcoreMesh(core_axis_name="c", subcore_axis_name="s")


@pl.kernel(
    mesh=_MESH,
    out_type=jax.ShapeDtypeStruct((_B, _D), jnp.float32),
    scratch_types=(
        [pltpu.VMEM((_CH,), jnp.int32) for _ in range(_NIB)]
        + [pltpu.VMEM((_CH, _D), jnp.float32) for _ in range(_NRB)]
        + [pltpu.SemaphoreType.DMA for _ in range(_NIB + 2 * _NRB)]
    ),
)
def _gather_sc(table_hbm, idx_hbm, out_hbm, *scratch):
    idx_v = scratch[:_NIB]
    rows_v = scratch[_NIB:_NIB + _NRB]
    sem_i = scratch[_NIB + _NRB:2 * _NIB + _NRB]
    sem_g = scratch[2 * _NIB + _NRB:2 * _NIB + 2 * _NRB]
    sem_s = scratch[2 * _NIB + 2 * _NRB:]

    wid = lax.axis_index("s") * _NC + lax.axis_index("c")

    def c_of(i):
        return wid + i * _NW

    def idx_copy(i):
        b = i % _NIB
        return pltpu.make_async_copy(idx_hbm.at[pl.ds(c_of(i) * _CH, _CH)],
                                     idx_v[b], sem_i[b])

    def gather_copy(i):
        b = i % _NRB
        return pltpu.make_async_copy(table_hbm.at[idx_v[i % _NIB]],
                                     rows_v[b], sem_g[b])

    def store_copy(i):
        b = i % _NRB
        return pltpu.make_async_copy(rows_v[b],
                                     out_hbm.at[pl.ds(c_of(i) * _CH, _CH)],
                                     sem_s[b])

    _K = 1

    idx_copy(0).start()
    idx_copy(1).start()

    for i in range(_ITERS + _K):
        if i >= _K:
            @pl.when(c_of(i - _K) < _NCHUNKS)
            def _complete(i=i):
                gather_copy(i - _K).wait()
                store_copy(i - _K).start()

        if i < _ITERS:
            @pl.when(c_of(i) < _NCHUNKS)
            def _launch(i=i):
                idx_copy(i).wait()
                if i >= _NRB:
                    store_copy(i - _NRB).wait()
                gather_copy(i).start()

            if i + 2 < _ITERS:
                @pl.when(c_of(i + 2) < _NCHUNKS)
                def _prefetch(i=i):
                    idx_copy(i + 2).start()

    for i in range(_ITERS - _NRB, _ITERS):
        @pl.when(c_of(i) < _NCHUNKS)
        def _drain(i=i):
            store_copy(i).wait()


def kernel(encoded_ast_nodes, symbol_leaf_nodes_indices, symbol_leaf_symbol_idx):
    out = _gather_sc(encoded_ast_nodes, symbol_leaf_nodes_indices)
    return (out, symbol_leaf_symbol_idx)


# speedup vs baseline: 1.0372x; 1.0372x over previous
import jax
import jax.numpy as jnp
from jax import lax
from jax.experimental import pallas as pl
from jax.experimental.pallas import tpu as pltpu
from jax.experimental.pallas import tpu_sc as plsc

_B = 100000
_D = 256
_CH = 160
_NCHUNKS = _B // _CH
_ITERS = -(-_NCHUNKS // 32)
_NRB = 3
_NIB = 4

_INFO = plsc.get_sparse_core_info()
_NC = _INFO.num_cores
_NW = _INFO.num_cores * _INFO.num_subcores

_MESH = plsc.VectorSubcoreMesh(core_axis_name="c", subcore_axis_name="s")


@pl.kernel(
    mesh=_MESH,
    out_type=jax.ShapeDtypeStruct((_B, _D), jnp.float32),
    scratch_types=(
        [pltpu.VMEM((_CH,), jnp.int32) for _ in range(_NIB)]
        + [pltpu.VMEM((_CH, _D), jnp.float32) for _ in range(_NRB)]
        + [pltpu.SemaphoreType.DMA for _ in range(_NIB + 2 * _NRB)]
    ),
)
def _gather_sc(table_hbm, idx_hbm, out_hbm, *scratch):
    idx_v = scratch[:_NIB]
    rows_v = scratch[_NIB:_NIB + _NRB]
    sem_i = scratch[_NIB + _NRB:2 * _NIB + _NRB]
    sem_g = scratch[2 * _NIB + _NRB:2 * _NIB + 2 * _NRB]
    sem_s = scratch[2 * _NIB + 2 * _NRB:]

    wid = lax.axis_index("s") * _NC + lax.axis_index("c")

    def c_of(i):
        return wid + i * _NW

    def idx_copy(i):
        b = i % _NIB
        return pltpu.make_async_copy(idx_hbm.at[pl.ds(c_of(i) * _CH, _CH)],
                                     idx_v[b], sem_i[b])

    def gather_copy(i):
        b = i % _NRB
        return pltpu.make_async_copy(table_hbm.at[idx_v[i % _NIB]],
                                     rows_v[b], sem_g[b])

    def store_copy(i):
        b = i % _NRB
        return pltpu.make_async_copy(rows_v[b],
                                     out_hbm.at[pl.ds(c_of(i) * _CH, _CH)],
                                     sem_s[b])

    _K = 2

    idx_copy(0).start()
    idx_copy(1).start()

    for i in range(_ITERS + _K):
        if i >= _K:
            @pl.when(c_of(i - _K) < _NCHUNKS)
            def _complete(i=i):
                gather_copy(i - _K).wait()
                store_copy(i - _K).start()

        if i < _ITERS:
            @pl.when(c_of(i) < _NCHUNKS)
            def _launch(i=i):
                idx_copy(i).wait()
                if i >= _NRB:
                    store_copy(i - _NRB).wait()
                gather_copy(i).start()

            if i + 2 < _ITERS:
                @pl.when(c_of(i + 2) < _NCHUNKS)
                def _prefetch(i=i):
                    idx_copy(i + 2).start()

    for i in range(_ITERS - _NRB, _ITERS):
        @pl.when(c_of(i) < _NCHUNKS)
        def _drain(i=i):
            store_copy(i).wait()


def kernel(encoded_ast_nodes, symbol_leaf_nodes_indices, symbol_leaf_symbol_idx):
    out = _gather_sc(encoded_ast_nodes, symbol_leaf_nodes_indices)
    return (out, symbol_leaf_symbol_idx)
